# baseline (speedup 1.0000x reference)
import functools

import jax
import jax.numpy as jnp
import numpy as np
from jax import lax
from jax.experimental import pallas as pl
from jax.experimental.pallas import tpu as pltpu
from jax.experimental.pallas import tpu_sc as plsc

_TABLE_SIZES = [100000] * 26
_NF = len(_TABLE_SIZES)
_OUT_DIM = 64
_BATCH = 16384
_B_FLAT = _BATCH * _NF
_NW = 32
_PER_W = _B_FLAT // _NW
_CHUNK = 128
_NCHUNK = _PER_W // _CHUNK
_NBUF = 8
_LAG = 4

_ROWS = sum(_TABLE_SIZES)
_RPAD = 2600064
_SLAB = 256
_NSLABS = _ROWS // _SLAB
_TAIL = _ROWS - _NSLABS * _SLAB

_OFFSETS = np.cumsum([0] + _TABLE_SIZES[:-1]).astype(np.int32)
_OFF_TILE = np.tile(_OFFSETS, _PER_W // _NF).reshape(_NCHUNK, _CHUNK)

_mesh = plsc.VectorSubcoreMesh(core_axis_name="c", subcore_axis_name="s")


@functools.partial(
    pl.kernel,
    mesh=_mesh,
    out_type=jax.ShapeDtypeStruct((_RPAD // 2, 2 * _OUT_DIM), jnp.float32),
    scratch_types=[
        pltpu.VMEM((2, _OUT_DIM, _SLAB), jnp.float32),
        pltpu.VMEM((2, _SLAB // 2, 2 * _OUT_DIM), jnp.float32),
    ]
    + [pltpu.SemaphoreType.DMA] * 4,
    compiler_params=pltpu.CompilerParams(use_tc_tiling_on_sc=True,
                                         needs_layout_passes=False),
)
def _sc_transpose(tt_hbm, tail_hbm, out_hbm, in_v, st_v, *sems):
    isems = sems[:2]
    osems = sems[2:]
    wid = lax.axis_index("s") * 2 + lax.axis_index("c")
    nsteps = _NSLABS // _NW + 1

    def _in_start(s, b):
        pltpu.async_copy(
            tt_hbm.at[slice(None),
                      pl.ds(pl.multiple_of(s * _SLAB, _SLAB), _SLAB)],
            in_v.at[b], isems[b])

    def _in_wait(s, b):
        pltpu.make_async_copy(
            tt_hbm.at[slice(None),
                      pl.ds(pl.multiple_of(s * _SLAB, _SLAB), _SLAB)],
            in_v.at[b], isems[b]).wait()

    def _out_start(s, b):
        pltpu.async_copy(
            st_v.at[b],
            out_hbm.at[pl.ds(pl.multiple_of(s * (_SLAB // 2), _SLAB // 2),
                             _SLAB // 2)], osems[b])

    def _out_wait(s, b):
        pltpu.make_async_copy(
            st_v.at[b],
            out_hbm.at[pl.ds(pl.multiple_of(s * (_SLAB // 2), _SLAB // 2),
                             _SLAB // 2)], osems[b]).wait()

    iota16 = lax.iota(jnp.int32, 16)

    def _transpose_rows(b, nwide):
        def _row(i, carry):
            for h in range(2):
                rl = 2 * i + h
                r_idx = jnp.broadcast_to(rl, (16,)).astype(jnp.int32)
                for k in range(_OUT_DIM // 16):
                    vals = plsc.load_gather(in_v.at[b],
                                            [k * 16 + iota16, r_idx])
                    st_v[b, i, pl.ds(h * _OUT_DIM + k * 16, 16)] = vals
            return carry

        lax.fori_loop(0, nwide, _row, 0)

    @pl.when(wid < _NSLABS)
    def _():
        _in_start(wid, 0)

    def _pair(p, carry):
        for half in range(2):
            t = p * 2 + half
            s_expr = t * _NW + wid
            b = half

            @pl.when(s_expr < _NSLABS)
            def _():
                sn = s_expr + _NW

                @pl.when(sn < _NSLABS)
                def _():
                    _in_start(sn, 1 - b)

                _in_wait(s_expr, b)

                @pl.when(t >= 2)
                def _():
                    _out_wait(s_expr - 2 * _NW, b)

                _transpose_rows(b, _SLAB // 2)
                _out_start(s_expr, b)

        return carry

    npairs = (nsteps + 1) // 2
    lax.fori_loop(0, npairs, _pair, 0)

    _out_wait(0, 0)
    _out_wait(0, 1)

    @pl.when(wid == 0)
    def _():
        pltpu.sync_copy(tail_hbm, in_v.at[0, slice(None), pl.ds(0, 128)])
        _transpose_rows(0, 64)
        pltpu.sync_copy(
            st_v.at[0, pl.ds(32, 32)],
            out_hbm.at[pl.ds((_ROWS - 128) // 2 + 32, 32)])


@functools.partial(
    pl.kernel,
    mesh=_mesh,
    out_type=jax.ShapeDtypeStruct((_B_FLAT, _OUT_DIM), jnp.float32),
    scratch_types=[
        pltpu.VMEM((_NCHUNK, _CHUNK), jnp.int32),
        pltpu.VMEM((_NCHUNK, _CHUNK), jnp.int32),
        pltpu.VMEM((_NBUF, _CHUNK, _OUT_DIM), jnp.float32),
    ]
    + [pltpu.SemaphoreType.DMA] * (2 * _NBUF),
    compiler_params=pltpu.CompilerParams(use_tc_tiling_on_sc=False),
)
def _sc_gather(idx_hbm, off_hbm, table_hbm, out_hbm, idx_v, off_v, rows_v,
               *sems):
    gsems, ssems = sems[:_NBUF], sems[_NBUF:]
    wid = lax.axis_index("s") * 2 + lax.axis_index("c")
    base = wid * _PER_W

    pltpu.sync_copy(idx_hbm.at[wid], idx_v)
    pltpu.sync_copy(off_hbm, off_v)

    def _shift(c, carry):
        for j in range(_CHUNK // 16):
            s = pl.ds(j * 16, 16)
            idx_v[c, s] = idx_v[c, s] + off_v[c, s]
        return carry

    lax.fori_loop(0, _NCHUNK, _shift, 0)

    def _g_start(c, b):
        pltpu.async_copy(table_hbm.at[idx_v.at[c]], rows_v.at[b], gsems[b])

    def _g_wait(c, b):
        pltpu.make_async_copy(table_hbm.at[idx_v.at[c]], rows_v.at[b],
                              gsems[b]).wait()

    def _s_start(c, b):
        pltpu.async_copy(
            rows_v.at[b], out_hbm.at[pl.ds(base + c * _CHUNK, _CHUNK)],
            ssems[b])

    def _s_wait(c, b):
        pltpu.make_async_copy(
            rows_v.at[b], out_hbm.at[pl.ds(base + c * _CHUNK, _CHUNK)],
            ssems[b]).wait()

    for b in range(_NBUF):
        _g_start(b, b)
    for b in range(_NBUF - _LAG):
        _g_wait(b, b)
        _s_start(b, b)

    def _group(g, carry):
        c0 = g * _NBUF
        for b in range(_NBUF):
            c = c0 + b
            _s_wait(c - _NBUF, b)
            _g_start(c, b)
            bl = (b - _LAG) % _NBUF
            _g_wait(c - _LAG, bl)
            _s_start(c - _LAG, bl)
        return carry

    lax.fori_loop(1, _NCHUNK // _NBUF, _group, 0)

    for i in range(_LAG):
        c = _NCHUNK - _LAG + i
        b = c % _NBUF
        _g_wait(c, b)
        _s_start(c, b)
    for i in range(_NBUF):
        c = _NCHUNK - _NBUF + i
        b = c % _NBUF
        _s_wait(c, b)


def kernel(indices, table):
    tt = table.T
    tail_t = table[_ROWS - 128:].T
    packed = _sc_transpose(tt, tail_t)
    lin = packed.reshape(_RPAD, _OUT_DIM)
    idx3 = indices.reshape(_NW, _NCHUNK, _CHUNK)
    off = jnp.asarray(_OFF_TILE)
    out = _sc_gather(idx3, off, lin)
    return out.reshape(_BATCH, _NF, _OUT_DIM)


# speedup vs baseline: 1.2098x; 1.2098x over previous
import functools

import jax
import jax.numpy as jnp
import numpy as np
from jax import lax
from jax.experimental import pallas as pl
from jax.experimental.pallas import tpu as pltpu
from jax.experimental.pallas import tpu_sc as plsc

_TABLE_SIZES = [100000] * 26
_NF = len(_TABLE_SIZES)
_OUT_DIM = 64
_BATCH = 16384
_B_FLAT = _BATCH * _NF
_NW = 32
_PER_W = _B_FLAT // _NW
_CHUNK = 128
_NCHUNK = _PER_W // _CHUNK
_NBUF = 8
_LAG = 4

_ROWS = sum(_TABLE_SIZES)
_RPAD = 2600064
_SLAB = 256
_NSLABS = _ROWS // _SLAB

_OFFSETS = np.cumsum([0] + _TABLE_SIZES[:-1]).astype(np.int32)
_OFF_TILE = np.tile(_OFFSETS, _PER_W // _NF).reshape(_NCHUNK, _CHUNK)

_mesh = plsc.VectorSubcoreMesh(core_axis_name="c", subcore_axis_name="s")


@functools.partial(
    pl.kernel,
    mesh=_mesh,
    out_type=jax.ShapeDtypeStruct((_RPAD // 2, 2 * _OUT_DIM), jnp.float32),
    scratch_types=[
        pltpu.VMEM((2, _OUT_DIM, _SLAB), jnp.float32),
        pltpu.VMEM((2, _SLAB // 2, 2 * _OUT_DIM), jnp.float32),
    ]
    + [pltpu.SemaphoreType.DMA] * 4,
    compiler_params=pltpu.CompilerParams(use_tc_tiling_on_sc=True,
                                         needs_layout_passes=False),
)
def _sc_transpose(tt_hbm, tail_hbm, out_hbm, in_v, st_v, *sems):
    isems = sems[:2]
    osems = sems[2:]
    wid = lax.axis_index("s") * 2 + lax.axis_index("c")
    nsteps = _NSLABS // _NW + 1

    def _in_start(s, b):
        pltpu.async_copy(
            tt_hbm.at[slice(None),
                      pl.ds(pl.multiple_of(s * _SLAB, _SLAB), _SLAB)],
            in_v.at[b], isems[b])

    def _in_wait(s, b):
        pltpu.make_async_copy(
            tt_hbm.at[slice(None),
                      pl.ds(pl.multiple_of(s * _SLAB, _SLAB), _SLAB)],
            in_v.at[b], isems[b]).wait()

    def _out_start(s, b):
        pltpu.async_copy(
            st_v.at[b],
            out_hbm.at[pl.ds(pl.multiple_of(s * (_SLAB // 2), _SLAB // 2),
                             _SLAB // 2)], osems[b])

    def _out_wait(s, b):
        pltpu.make_async_copy(
            st_v.at[b],
            out_hbm.at[pl.ds(pl.multiple_of(s * (_SLAB // 2), _SLAB // 2),
                             _SLAB // 2)], osems[b]).wait()

    iota16 = lax.iota(jnp.int32, 16)
    row_base = iota16 >> 1
    lane_base = (iota16 & 1) * _OUT_DIM

    def _transpose_rows(b, ngroups):
        def _dim(d, carry):
            lane_d = lane_base + d
            for g in range(ngroups):
                vals = in_v[b, d, pl.ds(g * 16, 16)]
                plsc.store_scatter(st_v.at[b],
                                   [row_base + g * 8, lane_d], vals)
            return carry

        lax.fori_loop(0, _OUT_DIM, _dim, 0)

    @pl.when(wid < _NSLABS)
    def _():
        _in_start(wid, 0)

    def _pair(p, carry):
        for half in range(2):
            t = p * 2 + half
            s_expr = t * _NW + wid
            b = half

            @pl.when(s_expr < _NSLABS)
            def _():
                sn = s_expr + _NW

                @pl.when(sn < _NSLABS)
                def _():
                    _in_start(sn, 1 - b)

                _in_wait(s_expr, b)

                @pl.when(t >= 2)
                def _():
                    _out_wait(s_expr - 2 * _NW, b)

                _transpose_rows(b, _SLAB // 16)
                _out_start(s_expr, b)

        return carry

    npairs = (nsteps + 1) // 2
    lax.fori_loop(0, npairs, _pair, 0)

    _out_wait(0, 0)
    _out_wait(0, 1)

    @pl.when(wid == 0)
    def _():
        pltpu.sync_copy(tail_hbm, in_v.at[0, slice(None), pl.ds(0, 128)])
        _transpose_rows(0, 128 // 16)
        pltpu.sync_copy(
            st_v.at[0, pl.ds(32, 32)],
            out_hbm.at[pl.ds((_ROWS - 128) // 2 + 32, 32)])


@functools.partial(
    pl.kernel,
    mesh=_mesh,
    out_type=jax.ShapeDtypeStruct((_B_FLAT, _OUT_DIM), jnp.float32),
    scratch_types=[
        pltpu.VMEM((_NCHUNK, _CHUNK), jnp.int32),
        pltpu.VMEM((_NCHUNK, _CHUNK), jnp.int32),
        pltpu.VMEM((_NBUF, _CHUNK, _OUT_DIM), jnp.float32),
    ]
    + [pltpu.SemaphoreType.DMA] * (2 * _NBUF),
    compiler_params=pltpu.CompilerParams(use_tc_tiling_on_sc=False),
)
def _sc_gather(idx_hbm, off_hbm, table_hbm, out_hbm, idx_v, off_v, rows_v,
               *sems):
    gsems, ssems = sems[:_NBUF], sems[_NBUF:]
    wid = lax.axis_index("s") * 2 + lax.axis_index("c")
    base = wid * _PER_W

    pltpu.sync_copy(idx_hbm.at[wid], idx_v)
    pltpu.sync_copy(off_hbm, off_v)

    def _shift(c, carry):
        for j in range(_CHUNK // 16):
            s = pl.ds(j * 16, 16)
            idx_v[c, s] = idx_v[c, s] + off_v[c, s]
        return carry

    lax.fori_loop(0, _NCHUNK, _shift, 0)

    def _g_start(c, b):
        pltpu.async_copy(table_hbm.at[idx_v.at[c]], rows_v.at[b], gsems[b])

    def _g_wait(c, b):
        pltpu.make_async_copy(table_hbm.at[idx_v.at[c]], rows_v.at[b],
                              gsems[b]).wait()

    def _s_start(c, b):
        pltpu.async_copy(
            rows_v.at[b], out_hbm.at[pl.ds(base + c * _CHUNK, _CHUNK)],
            ssems[b])

    def _s_wait(c, b):
        pltpu.make_async_copy(
            rows_v.at[b], out_hbm.at[pl.ds(base + c * _CHUNK, _CHUNK)],
            ssems[b]).wait()

    for b in range(_NBUF):
        _g_start(b, b)
    for b in range(_NBUF - _LAG):
        _g_wait(b, b)
        _s_start(b, b)

    def _group(g, carry):
        c0 = g * _NBUF
        for b in range(_NBUF):
            c = c0 + b
            _s_wait(c - _NBUF, b)
            _g_start(c, b)
            bl = (b - _LAG) % _NBUF
            _g_wait(c - _LAG, bl)
            _s_start(c - _LAG, bl)
        return carry

    lax.fori_loop(1, _NCHUNK // _NBUF, _group, 0)

    for i in range(_LAG):
        c = _NCHUNK - _LAG + i
        b = c % _NBUF
        _g_wait(c, b)
        _s_start(c, b)
    for i in range(_NBUF):
        c = _NCHUNK - _NBUF + i
        b = c % _NBUF
        _s_wait(c, b)


def kernel(indices, table):
    tt = table.T
    tail_t = table[_ROWS - 128:].T
    packed = _sc_transpose(tt, tail_t)
    lin = packed.reshape(_RPAD, _OUT_DIM)
    idx3 = indices.reshape(_NW, _NCHUNK, _CHUNK)
    off = jnp.asarray(_OFF_TILE)
    out = _sc_gather(idx3, off, lin)
    return out.reshape(_BATCH, _NF, _OUT_DIM)


# speedup vs baseline: 2.2254x; 1.8394x over previous
import functools

import jax
import jax.numpy as jnp
import numpy as np
from jax import lax
from jax.experimental import pallas as pl
from jax.experimental.pallas import tpu as pltpu
from jax.experimental.pallas import tpu_sc as plsc

_TABLE_SIZES = [100000] * 26
_NF = len(_TABLE_SIZES)
_OUT_DIM = 64
_BATCH = 16384
_B_FLAT = _BATCH * _NF
_NW = 32
_PER_W = _B_FLAT // _NW
_CHUNK = 128
_NCHUNK = _PER_W // _CHUNK
_NBUF = 8
_LAG = 4

assert _B_FLAT % _NW == 0 and _PER_W % _CHUNK == 0 and _PER_W % _NF == 0

_OFFSETS = np.cumsum([0] + _TABLE_SIZES[:-1]).astype(np.int32)
_OFF_TILE = np.tile(_OFFSETS, _PER_W // _NF).reshape(_NCHUNK, _CHUNK)

_mesh = plsc.VectorSubcoreMesh(core_axis_name="c", subcore_axis_name="s")


@functools.partial(
    pl.kernel,
    mesh=_mesh,
    out_type=jax.ShapeDtypeStruct((_B_FLAT, _OUT_DIM), jnp.float32),
    scratch_types=[
        pltpu.VMEM((_NCHUNK, _CHUNK), jnp.int32),
        pltpu.VMEM((_NCHUNK, _CHUNK), jnp.int32),
        pltpu.VMEM((_NBUF, _CHUNK, _OUT_DIM), jnp.float32),
    ]
    + [pltpu.SemaphoreType.DMA] * (2 * _NBUF),
    compiler_params=pltpu.CompilerParams(use_tc_tiling_on_sc=False),
)
def _sc_gather(idx_hbm, off_hbm, table_hbm, out_hbm, idx_v, off_v, rows_v,
               *sems):
    gsems, ssems = sems[:_NBUF], sems[_NBUF:]
    wid = lax.axis_index("s") * 2 + lax.axis_index("c")
    base = wid * _PER_W

    pltpu.sync_copy(idx_hbm.at[wid], idx_v)
    pltpu.sync_copy(off_hbm, off_v)

    def _shift(c, carry):
        for j in range(_CHUNK // 16):
            s = pl.ds(j * 16, 16)
            idx_v[c, s] = idx_v[c, s] + off_v[c, s]
        return carry

    lax.fori_loop(0, _NCHUNK, _shift, 0)

    def _g_start(c, b):
        pltpu.async_copy(table_hbm.at[idx_v.at[c]], rows_v.at[b], gsems[b])

    def _g_wait(c, b):
        pltpu.make_async_copy(table_hbm.at[idx_v.at[c]], rows_v.at[b],
                              gsems[b]).wait()

    def _s_start(c, b):
        pltpu.async_copy(
            rows_v.at[b], out_hbm.at[pl.ds(base + c * _CHUNK, _CHUNK)],
            ssems[b])

    def _s_wait(c, b):
        pltpu.make_async_copy(
            rows_v.at[b], out_hbm.at[pl.ds(base + c * _CHUNK, _CHUNK)],
            ssems[b]).wait()

    for b in range(_NBUF):
        _g_start(b, b)
    for b in range(_NBUF - _LAG):
        _g_wait(b, b)
        _s_start(b, b)

    def _group(g, carry):
        c0 = g * _NBUF
        for b in range(_NBUF):
            c = c0 + b
            _s_wait(c - _NBUF, b)
            _g_start(c, b)
            bl = (b - _LAG) % _NBUF
            _g_wait(c - _LAG, bl)
            _s_start(c - _LAG, bl)
        return carry

    lax.fori_loop(1, _NCHUNK // _NBUF, _group, 0)

    for i in range(_LAG):
        c = _NCHUNK - _LAG + i
        b = c % _NBUF
        _g_wait(c, b)
        _s_start(c, b)
    for i in range(_NBUF):
        c = _NCHUNK - _NBUF + i
        b = c % _NBUF
        _s_wait(c, b)


def kernel(indices, table):
    idx3 = indices.reshape(_NW, _NCHUNK, _CHUNK)
    off = jnp.asarray(_OFF_TILE)
    out = _sc_gather(idx3, off, table)
    return out.reshape(_BATCH, _NF, _OUT_DIM)
